# Initial kernel scaffold; baseline (speedup 1.0000x reference)
import jax
import jax.numpy as jnp
from jax.experimental import pallas as pl


def kernel(z, edge_index, batch, Wz, bz, W1, b1, W2, b2, Wo, bo):
    raise NotImplementedError("write your pallas kernel here")


# speedup vs baseline: 14.8704x; 14.8704x over previous
import functools

import jax
import jax.numpy as jnp
from jax import lax
from jax.experimental import pallas as pl
from jax.experimental.pallas import tpu as pltpu
from jax.experimental.pallas import tpu_sc as plsc

NB = 5
NN = 10000
NT = 50000
NE = 800000
NL = 128
NH = 64
NPAD = 51200
ROWS = NE // 128
HALF = ROWS // 2
NC, NS = 2, 16
RB = 1600
GD = NPAD // RB


def _mesh():
    return plsc.VectorSubcoreMesh(
        core_axis_name="c", subcore_axis_name="s",
        num_cores=NC, num_subcores=NS)


def _sc_degree_body(dst_hbm, zeros_hbm, out_hbm, idxb, ones, acc):
    c = lax.axis_index("c")
    s = lax.axis_index("s")
    for i in range(8):
        ones[pl.ds(16 * i, 16)] = jnp.full((16,), 1.0, jnp.float32)
    pltpu.sync_copy(zeros_hbm.at[pl.ds(s * 3200, 3200)],
                    acc.at[pl.ds(s * 3200, 3200)])
    plsc.subcore_barrier()

    def step(i, carry):
        r = c * HALF + s + 16 * i

        @pl.when(r < (c + 1) * HALF)
        def _():
            pltpu.sync_copy(dst_hbm.at[r], idxb)
            pltpu.sync_copy(ones, acc.at[idxb], add=True)
        return carry

    lax.fori_loop(0, (HALF + NS - 1) // NS, step, 0)
    plsc.subcore_barrier()
    pltpu.sync_copy(acc.at[pl.ds(s * 3200, 3200)],
                    out_hbm.at[c, pl.ds(s * 3200, 3200)])


def _sc_degree(dst2, zeros_n):
    f = pl.kernel(
        _sc_degree_body,
        out_type=jax.ShapeDtypeStruct((NC, NPAD), jnp.float32),
        mesh=_mesh(),
        scratch_types=[
            pltpu.VMEM((128,), jnp.int32),
            pltpu.VMEM((128,), jnp.float32),
            pltpu.VMEM_SHARED((NPAD,), jnp.float32),
        ])
    return f(dst2, zeros_n)


def _sc_conv1_body(src_hbm, dst_hbm, dinv_hbm, batch_hbm, zeros_hbm,
                   out_hbm, sbuf, dbuf, wbuf, bbuf, fidx, acc):
    c = lax.axis_index("c")
    s = lax.axis_index("s")
    pltpu.sync_copy(zeros_hbm.at[pl.ds(s * 25600, 25600)],
                    acc.at[pl.ds(s * 25600, 25600)])
    plsc.subcore_barrier()

    def step(i, carry):
        r = c * HALF + s + 16 * i

        @pl.when(r < (c + 1) * HALF)
        def _():
            pltpu.sync_copy(src_hbm.at[r], sbuf)
            pltpu.sync_copy(dst_hbm.at[r], dbuf)
            pltpu.sync_copy(dinv_hbm.at[sbuf], wbuf)
            pltpu.sync_copy(batch_hbm.at[sbuf], bbuf)
            for j in range(8):
                d16 = dbuf[pl.ds(16 * j, 16)]
                b16 = bbuf[pl.ds(16 * j, 16)]
                fidx[pl.ds(16 * j, 16)] = d16 * 8 + b16
            pltpu.sync_copy(wbuf, acc.at[fidx], add=True)
        return carry

    lax.fori_loop(0, (HALF + NS - 1) // NS, step, 0)
    plsc.subcore_barrier()
    pltpu.sync_copy(acc.at[pl.ds(s * 25600, 25600)],
                    out_hbm.at[c, pl.ds(s * 25600, 25600)])


def _sc_conv1(src2, dst2, dinv, batch_pad, zeros_s):
    f = pl.kernel(
        _sc_conv1_body,
        out_type=jax.ShapeDtypeStruct((NC, NPAD * 8), jnp.float32),
        mesh=_mesh(),
        scratch_types=[
            pltpu.VMEM((128,), jnp.int32),
            pltpu.VMEM((128,), jnp.int32),
            pltpu.VMEM((128,), jnp.float32),
            pltpu.VMEM((128,), jnp.int32),
            pltpu.VMEM((128,), jnp.int32),
            pltpu.VMEM_SHARED((NPAD * 8,), jnp.float32),
        ])
    return f(src2, dst2, dinv, batch_pad, zeros_s)


def _sc_conv2_body(src_hbm, dst_hbm, y0_hbm, y1_hbm, zeros_hbm,
                   out_hbm, sbuf, dbuf, msg, acc):
    c = lax.axis_index("c")
    s = lax.axis_index("s")
    pltpu.sync_copy(zeros_hbm.at[pl.ds(s * 3200, 3200), :],
                    acc.at[pl.ds(s * 3200, 3200), :])
    plsc.subcore_barrier()

    def step(i, carry):
        r = s + 16 * i

        @pl.when(r < ROWS)
        def _():
            pltpu.sync_copy(src_hbm.at[r], sbuf)
            pltpu.sync_copy(dst_hbm.at[r], dbuf)

            @pl.when(c == 0)
            def _():
                pltpu.sync_copy(y0_hbm.at[sbuf], msg)

            @pl.when(c == 1)
            def _():
                pltpu.sync_copy(y1_hbm.at[sbuf], msg)

            pltpu.sync_copy(msg, acc.at[dbuf], add=True)
        return carry

    lax.fori_loop(0, (ROWS + NS - 1) // NS, step, 0)
    plsc.subcore_barrier()
    pltpu.sync_copy(acc.at[pl.ds(s * 3200, 3200), :],
                    out_hbm.at[c, pl.ds(s * 3200, 3200), :])


def _sc_conv2(src2, dst2, y0, y1, zeros_m):
    f = pl.kernel(
        _sc_conv2_body,
        out_type=jax.ShapeDtypeStruct((NC, NPAD, 32), jnp.float32),
        mesh=_mesh(),
        compiler_params=pltpu.CompilerParams(use_tc_tiling_on_sc=False),
        scratch_types=[
            pltpu.VMEM((128,), jnp.int32),
            pltpu.VMEM((128,), jnp.int32),
            pltpu.VMEM((128, 32), jnp.float32),
            pltpu.VMEM_SHARED((NPAD, 32), jnp.float32),
        ])
    return f(src2, dst2, y0, y1, zeros_m)


def _tc_dinv_body(d_ref, o_ref):
    d = d_ref[0, :] + d_ref[1, :] + 1.0
    o_ref[...] = lax.rsqrt(d).reshape(1, NPAD)


def _tc_dinv(deg2):
    return pl.pallas_call(
        _tc_dinv_body,
        out_shape=jax.ShapeDtypeStruct((1, NPAD), jnp.float32),
    )(deg2)


def _tc_dense_body(S_ref, dv_ref, bt_ref, z_ref, Wz_ref, bz_ref,
                   W1_ref, b1_ref, W2_ref, b2_ref,
                   y0_ref, y1_ref, q_ref):
    zn = lax.dot_general(z_ref[...], Wz_ref[...],
                         (((1,), (1,)), ((), ()))) + bz_ref[...]
    xw1d = lax.dot_general(zn, W1_ref[...], (((1,), (1,)), ((), ())))
    S = S_ref[0] + S_ref[1]
    dv = dv_ref[0, 0, :]
    bt = bt_ref[0, 0, :]
    oh = (bt[:, None] == lax.broadcasted_iota(jnp.int32, (RB, 5), 1)
          ).astype(jnp.float32)
    S5 = S[:, :5] + dv[:, None] * oh
    out1 = dv[:, None] * jnp.dot(S5, xw1d) + b1_ref[...]
    h1 = jnp.maximum(out1, 0.0)
    xw2 = lax.dot_general(h1, W2_ref[...], (((1,), (1,)), ((), ())))
    y = dv[:, None] * xw2
    q_ref[...] = dv[:, None] * y + b2_ref[...]
    y0_ref[...] = y[:, :32]
    y1_ref[...] = y[:, 32:]


def _tc_dense(S3, dinv3, batch3, z, Wz, bz, W1, b1, W2, b2):
    return pl.pallas_call(
        _tc_dense_body,
        grid=(GD,),
        in_specs=[
            pl.BlockSpec((2, RB, 8), lambda i: (0, i, 0)),
            pl.BlockSpec((1, 1, RB), lambda i: (i, 0, 0)),
            pl.BlockSpec((1, 1, RB), lambda i: (i, 0, 0)),
            pl.BlockSpec((NB, NL), lambda i: (0, 0)),
            pl.BlockSpec((NH, NL), lambda i: (0, 0)),
            pl.BlockSpec((NH,), lambda i: (0,)),
            pl.BlockSpec((NH, NH), lambda i: (0, 0)),
            pl.BlockSpec((NH,), lambda i: (0,)),
            pl.BlockSpec((NH, NH), lambda i: (0, 0)),
            pl.BlockSpec((NH,), lambda i: (0,)),
        ],
        out_specs=[
            pl.BlockSpec((RB, 32), lambda i: (i, 0)),
            pl.BlockSpec((RB, 32), lambda i: (i, 0)),
            pl.BlockSpec((RB, NH), lambda i: (i, 0)),
        ],
        out_shape=[
            jax.ShapeDtypeStruct((NPAD, 32), jnp.float32),
            jax.ShapeDtypeStruct((NPAD, 32), jnp.float32),
            jax.ShapeDtypeStruct((NPAD, NH), jnp.float32),
        ],
    )(S3, dinv3, batch3, z, Wz, bz, W1, b1, W2, b2)


def _tc_final_body(seg_ref, q_ref, dv_ref, Wo_ref, bo_ref, o_ref):
    dv = dv_ref[0, 0, :]
    wo = Wo_ref[...]
    q = q_ref[...]
    h0 = jnp.maximum(dv[:, None] * seg_ref[0] + q[:, :32], 0.0)
    h1 = jnp.maximum(dv[:, None] * seg_ref[1] + q[:, 32:], 0.0)
    lin = jnp.dot(h0, wo[0, :32]) + jnp.dot(h1, wo[0, 32:]) + bo_ref[0]
    o_ref[...] = jnp.tanh(lin).reshape(1, 1, RB)


def _tc_final(seg2, q, dinv3, Wo, bo):
    return pl.pallas_call(
        _tc_final_body,
        grid=(GD,),
        in_specs=[
            pl.BlockSpec((2, RB, 32), lambda i: (0, i, 0)),
            pl.BlockSpec((RB, NH), lambda i: (i, 0)),
            pl.BlockSpec((1, 1, RB), lambda i: (i, 0, 0)),
            pl.BlockSpec((1, NH), lambda i: (0, 0)),
            pl.BlockSpec((1,), lambda i: (0,)),
        ],
        out_specs=pl.BlockSpec((1, 1, RB), lambda i: (i, 0, 0)),
        out_shape=jax.ShapeDtypeStruct((GD, 1, RB), jnp.float32),
    )(seg2, q, dinv3, Wo, bo)


def kernel(z, edge_index, batch, Wz, bz, W1, b1, W2, b2, Wo, bo):
    src2 = edge_index[0].reshape(ROWS, 128)
    dst2 = edge_index[1].reshape(ROWS, 128)
    zeros_n = jnp.zeros((NPAD,), jnp.float32)
    zeros_s = jnp.zeros((NPAD * 8,), jnp.float32)
    zeros_m = jnp.zeros((NPAD, 32), jnp.float32)
    batch_pad = jnp.concatenate(
        [batch, jnp.zeros((NPAD - NT,), jnp.int32)])

    deg2 = _sc_degree(dst2, zeros_n)
    dinv = _tc_dinv(deg2).reshape(NPAD)
    S2 = _sc_conv1(src2, dst2, dinv, batch_pad, zeros_s)
    S3 = S2.reshape(NC, NPAD, 8)
    dinv3 = dinv.reshape(GD, 1, RB)
    batch3 = batch_pad.reshape(GD, 1, RB)
    y0, y1, q = _tc_dense(S3, dinv3, batch3, z, Wz, bz, W1, b1, W2, b2)
    seg2 = _sc_conv2(src2, dst2, y0, y1, zeros_m)
    spin3 = _tc_final(seg2, q, dinv3, Wo, bo)
    return spin3.reshape(NPAD)[:NT].reshape(NB, NN)


# speedup vs baseline: 20.5954x; 1.3850x over previous
import functools

import jax
import jax.numpy as jnp
from jax import lax
from jax.experimental import pallas as pl
from jax.experimental.pallas import tpu as pltpu
from jax.experimental.pallas import tpu_sc as plsc

NB = 5
NN = 10000
NT = 50000
NE = 800000
NL = 128
NH = 64
NPAD = 51200
ROWS = 6272
HALF = ROWS // 2
TROWS = ROWS // 16
NBLK = TROWS // 2
NC, NS = 2, 16
RB = 1600
GD = NPAD // RB


def _mesh():
    return plsc.VectorSubcoreMesh(
        core_axis_name="c", subcore_axis_name="s",
        num_cores=NC, num_subcores=NS)


def _sc_degree_body(dst_hbm, zeros_hbm, out_hbm, idxb, ones, acc):
    c = lax.axis_index("c")
    s = lax.axis_index("s")
    for i in range(8):
        ones[pl.ds(16 * i, 16)] = jnp.full((16,), 1.0, jnp.float32)
    pltpu.sync_copy(zeros_hbm.at[pl.ds(s * 3200, 3200)],
                    acc.at[pl.ds(s * 3200, 3200)])
    plsc.subcore_barrier()

    def step(i, carry):
        r = c * HALF + s + 16 * i

        @pl.when(r < (c + 1) * HALF)
        def _():
            pltpu.sync_copy(dst_hbm.at[r], idxb)
            pltpu.sync_copy(ones, acc.at[idxb], add=True)
        return carry

    lax.fori_loop(0, (HALF + NS - 1) // NS, step, 0)
    plsc.subcore_barrier()
    pltpu.sync_copy(acc.at[pl.ds(s * 3200, 3200)],
                    out_hbm.at[c, pl.ds(s * 3200, 3200)])


def _sc_degree(dst2, zeros_n):
    f = pl.kernel(
        _sc_degree_body,
        out_type=jax.ShapeDtypeStruct((NC, NPAD), jnp.float32),
        mesh=_mesh(),
        scratch_types=[
            pltpu.VMEM((128,), jnp.int32),
            pltpu.VMEM((128,), jnp.float32),
            pltpu.VMEM_SHARED((NPAD,), jnp.float32),
        ])
    return f(dst2, zeros_n)


def _sc_conv1_body(src_hbm, dst_hbm, dinv_hbm, batch_hbm, zeros_hbm,
                   out_hbm, sbuf, dbuf, wbuf, bbuf, fidx, acc):
    c = lax.axis_index("c")
    s = lax.axis_index("s")
    pltpu.sync_copy(zeros_hbm.at[pl.ds(s * 25600, 25600)],
                    acc.at[pl.ds(s * 25600, 25600)])
    plsc.subcore_barrier()

    def step(i, carry):
        r = c * HALF + s + 16 * i

        @pl.when(r < (c + 1) * HALF)
        def _():
            pltpu.sync_copy(src_hbm.at[r], sbuf)
            pltpu.sync_copy(dst_hbm.at[r], dbuf)
            pltpu.sync_copy(dinv_hbm.at[sbuf], wbuf)
            pltpu.sync_copy(batch_hbm.at[sbuf], bbuf)
            for j in range(8):
                d16 = dbuf[pl.ds(16 * j, 16)]
                b16 = bbuf[pl.ds(16 * j, 16)]
                fidx[pl.ds(16 * j, 16)] = d16 * 8 + b16
            pltpu.sync_copy(wbuf, acc.at[fidx], add=True)
        return carry

    lax.fori_loop(0, (HALF + NS - 1) // NS, step, 0)
    plsc.subcore_barrier()
    pltpu.sync_copy(acc.at[pl.ds(s * 25600, 25600)],
                    out_hbm.at[c, pl.ds(s * 25600, 25600)])


def _sc_conv1(src2, dst2, dinv, batch_pad, zeros_s):
    f = pl.kernel(
        _sc_conv1_body,
        out_type=jax.ShapeDtypeStruct((NC, NPAD * 8), jnp.float32),
        mesh=_mesh(),
        scratch_types=[
            pltpu.VMEM((128,), jnp.int32),
            pltpu.VMEM((128,), jnp.int32),
            pltpu.VMEM((128,), jnp.float32),
            pltpu.VMEM((128,), jnp.int32),
            pltpu.VMEM((128,), jnp.int32),
            pltpu.VMEM_SHARED((NPAD * 8,), jnp.float32),
        ])
    return f(src2, dst2, dinv, batch_pad, zeros_s)


def _sc_conv2_body(src_hbm, dst_hbm, y_hbm, zeros_hbm, out_hbm,
                   si0, di0, si1, di1, m0, m1,
                   sem_i0, sem_i1, sem_g, sem_s, acc):
    c = lax.axis_index("c")
    s = lax.axis_index("s")
    base = s * TROWS
    yc = y_hbm.at[c]

    def fire_idx(b, sbuf, dbuf, sem):
        r = base + b * 2
        pltpu.async_copy(src_hbm.at[pl.ds(r, 2)], sbuf, sem)
        pltpu.async_copy(dst_hbm.at[pl.ds(r, 2)], dbuf, sem)

    fire_idx(0, si0, di0, sem_i0)
    fire_idx(1, si1, di1, sem_i1)
    pltpu.sync_copy(zeros_hbm.at[pl.ds(s * 3200, 3200), :],
                    acc.at[pl.ds(s * 3200, 3200), :])
    plsc.subcore_barrier()

    def process(b, sbuf, dbuf, msg, sem_i):
        pltpu.make_async_copy(src_hbm.at[pl.ds(0, 2)], sbuf, sem_i).wait()
        pltpu.make_async_copy(dst_hbm.at[pl.ds(0, 2)], dbuf, sem_i).wait()
        gs = [pltpu.async_copy(yc.at[sbuf.at[j]], msg.at[j], sem_g)
              for j in range(2)]
        scs = []
        for j in range(2):
            gs[j].wait()
            scs.append(pltpu.async_copy(msg.at[j], acc.at[dbuf.at[j]],
                                        sem_s, add=True))
        for d in scs:
            d.wait()
        fire_idx(lax.rem(b + 2, NBLK), sbuf, dbuf, sem_i)

    def outer(i2, carry):
        process(2 * i2, si0, di0, m0, sem_i0)
        process(2 * i2 + 1, si1, di1, m1, sem_i1)
        return carry

    lax.fori_loop(0, NBLK // 2, outer, 0)
    for sbuf, dbuf, sem in ((si0, di0, sem_i0), (si1, di1, sem_i1)):
        pltpu.make_async_copy(src_hbm.at[pl.ds(0, 2)], sbuf, sem).wait()
        pltpu.make_async_copy(dst_hbm.at[pl.ds(0, 2)], dbuf, sem).wait()
    plsc.subcore_barrier()
    pltpu.sync_copy(acc.at[pl.ds(s * 3200, 3200), :],
                    out_hbm.at[c, pl.ds(s * 3200, 3200), :])


def _sc_conv2(src2, dst2, y2, zeros_m):
    f = pl.kernel(
        _sc_conv2_body,
        out_type=jax.ShapeDtypeStruct((NC, NPAD, 32), jnp.float32),
        mesh=_mesh(),
        compiler_params=pltpu.CompilerParams(use_tc_tiling_on_sc=False),
        scratch_types=[
            pltpu.VMEM((2, 128), jnp.int32),
            pltpu.VMEM((2, 128), jnp.int32),
            pltpu.VMEM((2, 128), jnp.int32),
            pltpu.VMEM((2, 128), jnp.int32),
            pltpu.VMEM((2, 128, 32), jnp.float32),
            pltpu.VMEM((2, 128, 32), jnp.float32),
            pltpu.SemaphoreType.DMA,
            pltpu.SemaphoreType.DMA,
            pltpu.SemaphoreType.DMA,
            pltpu.SemaphoreType.DMA,
            pltpu.VMEM_SHARED((NPAD, 32), jnp.float32),
        ])
    return f(src2, dst2, y2, zeros_m)


def _tc_dinv_body(d_ref, o_ref):
    d = d_ref[0, :] + d_ref[1, :] + 1.0
    o_ref[...] = lax.rsqrt(d).reshape(1, NPAD)


def _tc_dinv(deg2):
    return pl.pallas_call(
        _tc_dinv_body,
        out_shape=jax.ShapeDtypeStruct((1, NPAD), jnp.float32),
    )(deg2)


def _tc_dense_body(S_ref, dv_ref, bt_ref, z_ref, Wz_ref, bz_ref,
                   W1_ref, b1_ref, W2_ref, b2_ref,
                   y2_ref, q_ref):
    bf = jnp.bfloat16
    f32 = jnp.float32
    zn = lax.dot_general(z_ref[...].astype(bf), Wz_ref[...].astype(bf),
                         (((1,), (1,)), ((), ())),
                         preferred_element_type=f32) + bz_ref[...]
    xw1d = lax.dot_general(zn.astype(bf), W1_ref[...].astype(bf),
                           (((1,), (1,)), ((), ())),
                           preferred_element_type=f32)
    S = S_ref[0] + S_ref[1]
    dv = dv_ref[0, 0, :]
    bt = bt_ref[0, 0, :]
    oh = (bt[:, None] == lax.broadcasted_iota(jnp.int32, (RB, 5), 1)
          ).astype(jnp.float32)
    S5 = S[:, :5] + dv[:, None] * oh
    t = S5[:, 0:1] * xw1d[0:1, :]
    for b in range(1, 5):
        t = t + S5[:, b:b + 1] * xw1d[b:b + 1, :]
    out1 = dv[:, None] * t + b1_ref[...]
    h1 = jnp.maximum(out1, 0.0)
    xw2 = lax.dot_general(h1.astype(bf), W2_ref[...].astype(bf),
                          (((1,), (1,)), ((), ())),
                          preferred_element_type=f32)
    y = dv[:, None] * xw2
    q_ref[...] = dv[:, None] * y + b2_ref[...]
    y2_ref[0] = y[:, :32]
    y2_ref[1] = y[:, 32:]


def _tc_dense(S3, dinv3, batch3, z, Wz, bz, W1, b1, W2, b2):
    return pl.pallas_call(
        _tc_dense_body,
        grid=(GD,),
        in_specs=[
            pl.BlockSpec((2, RB, 8), lambda i: (0, i, 0)),
            pl.BlockSpec((1, 1, RB), lambda i: (i, 0, 0)),
            pl.BlockSpec((1, 1, RB), lambda i: (i, 0, 0)),
            pl.BlockSpec((NB, NL), lambda i: (0, 0)),
            pl.BlockSpec((NH, NL), lambda i: (0, 0)),
            pl.BlockSpec((NH,), lambda i: (0,)),
            pl.BlockSpec((NH, NH), lambda i: (0, 0)),
            pl.BlockSpec((NH,), lambda i: (0,)),
            pl.BlockSpec((NH, NH), lambda i: (0, 0)),
            pl.BlockSpec((NH,), lambda i: (0,)),
        ],
        out_specs=[
            pl.BlockSpec((2, RB, 32), lambda i: (0, i, 0)),
            pl.BlockSpec((RB, NH), lambda i: (i, 0)),
        ],
        out_shape=[
            jax.ShapeDtypeStruct((2, NPAD, 32), jnp.float32),
            jax.ShapeDtypeStruct((NPAD, NH), jnp.float32),
        ],
    )(S3, dinv3, batch3, z, Wz, bz, W1, b1, W2, b2)


def _tc_final_body(seg_ref, q_ref, dv_ref, Wo_ref, bo_ref, o_ref):
    dv = dv_ref[0, 0, :]
    wo = Wo_ref[...]
    q = q_ref[...]
    h0 = jnp.maximum(dv[:, None] * seg_ref[0] + q[:, :32], 0.0)
    h1 = jnp.maximum(dv[:, None] * seg_ref[1] + q[:, 32:], 0.0)
    bf = jnp.bfloat16
    f32 = jnp.float32
    lin = (jnp.dot(h0.astype(bf), wo[0, :32].astype(bf),
                   preferred_element_type=f32)
           + jnp.dot(h1.astype(bf), wo[0, 32:].astype(bf),
                     preferred_element_type=f32) + bo_ref[0])
    o_ref[...] = jnp.tanh(lin).reshape(1, 1, RB)


def _tc_final(seg2, q, dinv3, Wo, bo):
    return pl.pallas_call(
        _tc_final_body,
        grid=(GD,),
        in_specs=[
            pl.BlockSpec((2, RB, 32), lambda i: (0, i, 0)),
            pl.BlockSpec((RB, NH), lambda i: (i, 0)),
            pl.BlockSpec((1, 1, RB), lambda i: (i, 0, 0)),
            pl.BlockSpec((1, NH), lambda i: (0, 0)),
            pl.BlockSpec((1,), lambda i: (0,)),
        ],
        out_specs=pl.BlockSpec((1, 1, RB), lambda i: (i, 0, 0)),
        out_shape=jax.ShapeDtypeStruct((GD, 1, RB), jnp.float32),
    )(seg2, q, dinv3, Wo, bo)


def kernel(z, edge_index, batch, Wz, bz, W1, b1, W2, b2, Wo, bo):
    npad_e = ROWS * 128 - NE
    pad = jnp.full((npad_e,), NT, jnp.int32)
    src2 = jnp.concatenate([edge_index[0], pad]).reshape(ROWS, 128)
    dst2 = jnp.concatenate([edge_index[1], pad]).reshape(ROWS, 128)
    zeros_n = jnp.zeros((NPAD,), jnp.float32)
    zeros_s = jnp.zeros((NPAD * 8,), jnp.float32)
    zeros_m = jnp.zeros((NPAD, 32), jnp.float32)
    batch_pad = jnp.concatenate(
        [batch, jnp.zeros((NPAD - NT,), jnp.int32)])

    deg2 = _sc_degree(dst2, zeros_n)
    dinv = _tc_dinv(deg2).reshape(NPAD)
    S2 = _sc_conv1(src2, dst2, dinv, batch_pad, zeros_s)
    S3 = S2.reshape(NC, NPAD, 8)
    dinv3 = dinv.reshape(GD, 1, RB)
    batch3 = batch_pad.reshape(GD, 1, RB)
    y2, q = _tc_dense(S3, dinv3, batch3, z, Wz, bz, W1, b1, W2, b2)
    seg2 = _sc_conv2(src2, dst2, y2, zeros_m)
    spin3 = _tc_final(seg2, q, dinv3, Wo, bo)
    return spin3.reshape(NPAD)[:NT].reshape(NB, NN)


# speedup vs baseline: 32.0688x; 1.5571x over previous
import functools

import jax
import jax.numpy as jnp
from jax import lax
from jax.experimental import pallas as pl
from jax.experimental.pallas import tpu as pltpu
from jax.experimental.pallas import tpu_sc as plsc

NB = 5
NN = 10000
NT = 50000
NE = 800000
NL = 128
NH = 64
NPAD = 51200
ROWS = 6272
HALF = ROWS // 2
TROWS = ROWS // 16
NBLK = TROWS // 2
NC, NS = 2, 16
RB = 1600
GD = NPAD // RB


def _mesh():
    return plsc.VectorSubcoreMesh(
        core_axis_name="c", subcore_axis_name="s",
        num_cores=NC, num_subcores=NS)


TBLK = HALF // NS // 2


def _sc_degree_body(dst_hbm, zeros_hbm, out_hbm, di0, di1, ones,
                    sem_i0, sem_i1, sem_s, acc):
    c = lax.axis_index("c")
    s = lax.axis_index("s")
    base = c * HALF + s * (2 * TBLK)
    for i in range(8):
        ones[pl.ds(16 * i, 16)] = jnp.full((16,), 1.0, jnp.float32)

    def fire_idx(b, dbuf, sem):
        pltpu.async_copy(dst_hbm.at[pl.ds(base + b * 2, 2)], dbuf, sem)

    fire_idx(0, di0, sem_i0)
    fire_idx(1, di1, sem_i1)
    pltpu.sync_copy(zeros_hbm.at[pl.ds(s * 3200, 3200)],
                    acc.at[pl.ds(s * 3200, 3200)])
    plsc.subcore_barrier()

    def process(b, dbuf, sem_i):
        pltpu.make_async_copy(dst_hbm.at[pl.ds(0, 2)], dbuf, sem_i).wait()
        scs = [pltpu.async_copy(ones, acc.at[dbuf.at[j]], sem_s, add=True)
               for j in range(2)]
        for d in scs:
            d.wait()
        fire_idx(lax.rem(b + 2, TBLK), dbuf, sem_i)

    def outer(i2, carry):
        process(2 * i2, di0, sem_i0)
        process(2 * i2 + 1, di1, sem_i1)
        return carry

    lax.fori_loop(0, TBLK // 2, outer, 0)
    for dbuf, sem in ((di0, sem_i0), (di1, sem_i1)):
        pltpu.make_async_copy(dst_hbm.at[pl.ds(0, 2)], dbuf, sem).wait()
    plsc.subcore_barrier()
    pltpu.sync_copy(acc.at[pl.ds(s * 3200, 3200)],
                    out_hbm.at[c, pl.ds(s * 3200, 3200)])


def _sc_degree(dst2, zeros_n):
    f = pl.kernel(
        _sc_degree_body,
        out_type=jax.ShapeDtypeStruct((NC, NPAD), jnp.float32),
        mesh=_mesh(),
        scratch_types=[
            pltpu.VMEM((2, 128), jnp.int32),
            pltpu.VMEM((2, 128), jnp.int32),
            pltpu.VMEM((128,), jnp.float32),
            pltpu.SemaphoreType.DMA,
            pltpu.SemaphoreType.DMA,
            pltpu.SemaphoreType.DMA,
            pltpu.VMEM_SHARED((NPAD,), jnp.float32),
        ])
    return f(dst2, zeros_n)


def _sc_conv1_body(src_hbm, dst_hbm, dinv_hbm, batch_hbm, zeros_hbm,
                   out_hbm, si0, di0, si1, di1, wv0, bv0, wv1, bv1,
                   fx0, fx1, sem_i0, sem_i1, sem_g, sem_s, acc):
    c = lax.axis_index("c")
    s = lax.axis_index("s")
    base = c * HALF + s * (2 * TBLK)

    def fire_idx(b, sbuf, dbuf, sem):
        r = base + b * 2
        pltpu.async_copy(src_hbm.at[pl.ds(r, 2)], sbuf, sem)
        pltpu.async_copy(dst_hbm.at[pl.ds(r, 2)], dbuf, sem)

    fire_idx(0, si0, di0, sem_i0)
    fire_idx(1, si1, di1, sem_i1)
    pltpu.sync_copy(zeros_hbm.at[pl.ds(s * 25600, 25600)],
                    acc.at[pl.ds(s * 25600, 25600)])
    plsc.subcore_barrier()

    def process(b, sbuf, dbuf, wv, bv, fx, sem_i):
        pltpu.make_async_copy(src_hbm.at[pl.ds(0, 2)], sbuf, sem_i).wait()
        pltpu.make_async_copy(dst_hbm.at[pl.ds(0, 2)], dbuf, sem_i).wait()
        gs = []
        for j in range(2):
            gs.append(pltpu.async_copy(dinv_hbm.at[sbuf.at[j]],
                                       wv.at[j], sem_g))
            gs.append(pltpu.async_copy(batch_hbm.at[sbuf.at[j]],
                                       bv.at[j], sem_g))
        for d in gs:
            d.wait()
        for j in range(2):
            for k in range(8):
                d16 = dbuf[j, pl.ds(16 * k, 16)]
                b16 = bv[j, pl.ds(16 * k, 16)]
                fx[j, pl.ds(16 * k, 16)] = d16 * 8 + b16
        scs = [pltpu.async_copy(wv.at[j], acc.at[fx.at[j]],
                                sem_s, add=True)
               for j in range(2)]
        for d in scs:
            d.wait()
        fire_idx(lax.rem(b + 2, TBLK), sbuf, dbuf, sem_i)

    def outer(i2, carry):
        process(2 * i2, si0, di0, wv0, bv0, fx0, sem_i0)
        process(2 * i2 + 1, si1, di1, wv1, bv1, fx1, sem_i1)
        return carry

    lax.fori_loop(0, TBLK // 2, outer, 0)
    for sbuf, dbuf, sem in ((si0, di0, sem_i0), (si1, di1, sem_i1)):
        pltpu.make_async_copy(src_hbm.at[pl.ds(0, 2)], sbuf, sem).wait()
        pltpu.make_async_copy(dst_hbm.at[pl.ds(0, 2)], dbuf, sem).wait()
    plsc.subcore_barrier()
    pltpu.sync_copy(acc.at[pl.ds(s * 25600, 25600)],
                    out_hbm.at[c, pl.ds(s * 25600, 25600)])


def _sc_conv1(src2, dst2, dinv, batch_pad, zeros_s):
    f = pl.kernel(
        _sc_conv1_body,
        out_type=jax.ShapeDtypeStruct((NC, NPAD * 8), jnp.float32),
        mesh=_mesh(),
        scratch_types=[
            pltpu.VMEM((2, 128), jnp.int32),
            pltpu.VMEM((2, 128), jnp.int32),
            pltpu.VMEM((2, 128), jnp.int32),
            pltpu.VMEM((2, 128), jnp.int32),
            pltpu.VMEM((2, 128), jnp.float32),
            pltpu.VMEM((2, 128), jnp.int32),
            pltpu.VMEM((2, 128), jnp.float32),
            pltpu.VMEM((2, 128), jnp.int32),
            pltpu.VMEM((2, 128), jnp.int32),
            pltpu.VMEM((2, 128), jnp.int32),
            pltpu.SemaphoreType.DMA,
            pltpu.SemaphoreType.DMA,
            pltpu.SemaphoreType.DMA,
            pltpu.SemaphoreType.DMA,
            pltpu.VMEM_SHARED((NPAD * 8,), jnp.float32),
        ])
    return f(src2, dst2, dinv, batch_pad, zeros_s)


def _sc_conv2_body(src_hbm, dst_hbm, y_hbm, zeros_hbm, out_hbm,
                   si0, di0, si1, di1, m0, m1,
                   sem_i0, sem_i1, sem_g, sem_s, acc):
    c = lax.axis_index("c")
    s = lax.axis_index("s")
    base = s * TROWS
    yc = y_hbm.at[c]

    def fire_idx(b, sbuf, dbuf, sem):
        r = base + b * 2
        pltpu.async_copy(src_hbm.at[pl.ds(r, 2)], sbuf, sem)
        pltpu.async_copy(dst_hbm.at[pl.ds(r, 2)], dbuf, sem)

    fire_idx(0, si0, di0, sem_i0)
    fire_idx(1, si1, di1, sem_i1)
    pltpu.sync_copy(zeros_hbm.at[pl.ds(s * 3200, 3200), :],
                    acc.at[pl.ds(s * 3200, 3200), :])
    plsc.subcore_barrier()

    def process(b, sbuf, dbuf, msg, sem_i):
        pltpu.make_async_copy(src_hbm.at[pl.ds(0, 2)], sbuf, sem_i).wait()
        pltpu.make_async_copy(dst_hbm.at[pl.ds(0, 2)], dbuf, sem_i).wait()
        gs = [pltpu.async_copy(yc.at[sbuf.at[j]], msg.at[j], sem_g)
              for j in range(2)]
        scs = []
        for j in range(2):
            gs[j].wait()
            scs.append(pltpu.async_copy(msg.at[j], acc.at[dbuf.at[j]],
                                        sem_s, add=True))
        for d in scs:
            d.wait()
        fire_idx(lax.rem(b + 2, NBLK), sbuf, dbuf, sem_i)

    def outer(i2, carry):
        process(2 * i2, si0, di0, m0, sem_i0)
        process(2 * i2 + 1, si1, di1, m1, sem_i1)
        return carry

    lax.fori_loop(0, NBLK // 2, outer, 0)
    for sbuf, dbuf, sem in ((si0, di0, sem_i0), (si1, di1, sem_i1)):
        pltpu.make_async_copy(src_hbm.at[pl.ds(0, 2)], sbuf, sem).wait()
        pltpu.make_async_copy(dst_hbm.at[pl.ds(0, 2)], dbuf, sem).wait()
    plsc.subcore_barrier()
    pltpu.sync_copy(acc.at[pl.ds(s * 3200, 3200), :],
                    out_hbm.at[c, pl.ds(s * 3200, 3200), :])


def _sc_conv2(src2, dst2, y2, zeros_m):
    f = pl.kernel(
        _sc_conv2_body,
        out_type=jax.ShapeDtypeStruct((NC, NPAD, 32), jnp.float32),
        mesh=_mesh(),
        compiler_params=pltpu.CompilerParams(use_tc_tiling_on_sc=False),
        scratch_types=[
            pltpu.VMEM((2, 128), jnp.int32),
            pltpu.VMEM((2, 128), jnp.int32),
            pltpu.VMEM((2, 128), jnp.int32),
            pltpu.VMEM((2, 128), jnp.int32),
            pltpu.VMEM((2, 128, 32), jnp.float32),
            pltpu.VMEM((2, 128, 32), jnp.float32),
            pltpu.SemaphoreType.DMA,
            pltpu.SemaphoreType.DMA,
            pltpu.SemaphoreType.DMA,
            pltpu.SemaphoreType.DMA,
            pltpu.VMEM_SHARED((NPAD, 32), jnp.float32),
        ])
    return f(src2, dst2, y2, zeros_m)


def _tc_dinv_body(d_ref, o_ref):
    d = d_ref[0, :] + d_ref[1, :] + 1.0
    o_ref[...] = lax.rsqrt(d).reshape(1, NPAD)


def _tc_dinv(deg2):
    return pl.pallas_call(
        _tc_dinv_body,
        out_shape=jax.ShapeDtypeStruct((1, NPAD), jnp.float32),
    )(deg2)


def _tc_dense_body(S_ref, dv_ref, bt_ref, z_ref, Wz_ref, bz_ref,
                   W1_ref, b1_ref, W2_ref, b2_ref,
                   y2_ref, q_ref):
    bf = jnp.bfloat16
    f32 = jnp.float32
    zn = lax.dot_general(z_ref[...].astype(bf), Wz_ref[...].astype(bf),
                         (((1,), (1,)), ((), ())),
                         preferred_element_type=f32) + bz_ref[...]
    xw1d = lax.dot_general(zn.astype(bf), W1_ref[...].astype(bf),
                           (((1,), (1,)), ((), ())),
                           preferred_element_type=f32)
    S = S_ref[0] + S_ref[1]
    dv = dv_ref[0, 0, :]
    bt = bt_ref[0, 0, :]
    oh = (bt[:, None] == lax.broadcasted_iota(jnp.int32, (RB, 5), 1)
          ).astype(jnp.float32)
    S5 = S[:, :5] + dv[:, None] * oh
    t = S5[:, 0:1] * xw1d[0:1, :]
    for b in range(1, 5):
        t = t + S5[:, b:b + 1] * xw1d[b:b + 1, :]
    out1 = dv[:, None] * t + b1_ref[...]
    h1 = jnp.maximum(out1, 0.0)
    xw2 = lax.dot_general(h1.astype(bf), W2_ref[...].astype(bf),
                          (((1,), (1,)), ((), ())),
                          preferred_element_type=f32)
    y = dv[:, None] * xw2
    q_ref[...] = dv[:, None] * y + b2_ref[...]
    y2_ref[0] = y[:, :32]
    y2_ref[1] = y[:, 32:]


def _tc_dense(S3, dinv3, batch3, z, Wz, bz, W1, b1, W2, b2):
    return pl.pallas_call(
        _tc_dense_body,
        grid=(GD,),
        in_specs=[
            pl.BlockSpec((2, RB, 8), lambda i: (0, i, 0)),
            pl.BlockSpec((1, 1, RB), lambda i: (i, 0, 0)),
            pl.BlockSpec((1, 1, RB), lambda i: (i, 0, 0)),
            pl.BlockSpec((NB, NL), lambda i: (0, 0)),
            pl.BlockSpec((NH, NL), lambda i: (0, 0)),
            pl.BlockSpec((NH,), lambda i: (0,)),
            pl.BlockSpec((NH, NH), lambda i: (0, 0)),
            pl.BlockSpec((NH,), lambda i: (0,)),
            pl.BlockSpec((NH, NH), lambda i: (0, 0)),
            pl.BlockSpec((NH,), lambda i: (0,)),
        ],
        out_specs=[
            pl.BlockSpec((2, RB, 32), lambda i: (0, i, 0)),
            pl.BlockSpec((RB, NH), lambda i: (i, 0)),
        ],
        out_shape=[
            jax.ShapeDtypeStruct((2, NPAD, 32), jnp.float32),
            jax.ShapeDtypeStruct((NPAD, NH), jnp.float32),
        ],
    )(S3, dinv3, batch3, z, Wz, bz, W1, b1, W2, b2)


def _tc_final_body(seg_ref, q_ref, dv_ref, Wo_ref, bo_ref, o_ref):
    dv = dv_ref[0, 0, :]
    wo = Wo_ref[...]
    q = q_ref[...]
    h0 = jnp.maximum(dv[:, None] * seg_ref[0] + q[:, :32], 0.0)
    h1 = jnp.maximum(dv[:, None] * seg_ref[1] + q[:, 32:], 0.0)
    bf = jnp.bfloat16
    f32 = jnp.float32
    lin = (jnp.dot(h0.astype(bf), wo[0, :32].astype(bf),
                   preferred_element_type=f32)
           + jnp.dot(h1.astype(bf), wo[0, 32:].astype(bf),
                     preferred_element_type=f32) + bo_ref[0])
    o_ref[...] = jnp.tanh(lin).reshape(1, 1, RB)


def _tc_final(seg2, q, dinv3, Wo, bo):
    return pl.pallas_call(
        _tc_final_body,
        grid=(GD,),
        in_specs=[
            pl.BlockSpec((2, RB, 32), lambda i: (0, i, 0)),
            pl.BlockSpec((RB, NH), lambda i: (i, 0)),
            pl.BlockSpec((1, 1, RB), lambda i: (i, 0, 0)),
            pl.BlockSpec((1, NH), lambda i: (0, 0)),
            pl.BlockSpec((1,), lambda i: (0,)),
        ],
        out_specs=pl.BlockSpec((1, 1, RB), lambda i: (i, 0, 0)),
        out_shape=jax.ShapeDtypeStruct((GD, 1, RB), jnp.float32),
    )(seg2, q, dinv3, Wo, bo)


def kernel(z, edge_index, batch, Wz, bz, W1, b1, W2, b2, Wo, bo):
    npad_e = ROWS * 128 - NE
    pad = jnp.full((npad_e,), NT, jnp.int32)
    src2 = jnp.concatenate([edge_index[0], pad]).reshape(ROWS, 128)
    dst2 = jnp.concatenate([edge_index[1], pad]).reshape(ROWS, 128)
    zeros_n = jnp.zeros((NPAD,), jnp.float32)
    zeros_s = jnp.zeros((NPAD * 8,), jnp.float32)
    zeros_m = jnp.zeros((NPAD, 32), jnp.float32)
    batch_pad = jnp.concatenate(
        [batch, jnp.zeros((NPAD - NT,), jnp.int32)])

    deg2 = _sc_degree(dst2, zeros_n)
    dinv = _tc_dinv(deg2).reshape(NPAD)
    S2 = _sc_conv1(src2, dst2, dinv, batch_pad, zeros_s)
    S3 = S2.reshape(NC, NPAD, 8)
    dinv3 = dinv.reshape(GD, 1, RB)
    batch3 = batch_pad.reshape(GD, 1, RB)
    y2, q = _tc_dense(S3, dinv3, batch3, z, Wz, bz, W1, b1, W2, b2)
    seg2 = _sc_conv2(src2, dst2, y2, zeros_m)
    spin3 = _tc_final(seg2, q, dinv3, Wo, bo)
    return spin3.reshape(NPAD)[:NT].reshape(NB, NN)


# speedup vs baseline: 38.4344x; 1.1985x over previous
import functools

import jax
import jax.numpy as jnp
from jax import lax
from jax.experimental import pallas as pl
from jax.experimental.pallas import tpu as pltpu
from jax.experimental.pallas import tpu_sc as plsc

NB = 5
NN = 10000
NT = 50000
NE = 800000
NL = 128
NH = 64
NPAD = 51200
ROWS = 6272
HALF = ROWS // 2
TROWS = ROWS // 16
NBLK = TROWS // 2
NC, NS = 2, 16
RB = 1600
GD = NPAD // RB


def _mesh():
    return plsc.VectorSubcoreMesh(
        core_axis_name="c", subcore_axis_name="s",
        num_cores=NC, num_subcores=NS)


TBLKD = (ROWS // NS) // 2
TBLK = (HALF // NS) // 2


def _sc_prep_body(src_hbm, dst_hbm, batch_hbm, zn_hbm, zs_hbm,
                  s_out, pk_out,
                  si0, si1, di0, di1, ones, db, pb,
                  pv0, pv1, wv0, wv1, fx0, fx1,
                  sem_i0, sem_i1, sem_g, sem_s, acc_n, acc_s):
    c = lax.axis_index("c")
    s = lax.axis_index("s")
    for i in range(8):
        ones[pl.ds(16 * i, 16)] = jnp.full((16,), 1, jnp.int32)

    based = s * (2 * TBLKD)

    def fire_d(b, dbuf, sem):
        pltpu.async_copy(dst_hbm.at[pl.ds(based + b * 2, 2)], dbuf, sem)

    fire_d(0, di0, sem_i0)
    fire_d(1, di1, sem_i1)
    pltpu.sync_copy(zn_hbm.at[pl.ds(s * 3200, 3200)],
                    acc_n.at[pl.ds(s * 3200, 3200)])
    pltpu.sync_copy(zs_hbm.at[pl.ds(s * 25600, 25600)],
                    acc_s.at[pl.ds(s * 25600, 25600)])
    plsc.subcore_barrier()

    def proc_d(b, dbuf, sem_i):
        pltpu.make_async_copy(dst_hbm.at[pl.ds(0, 2)], dbuf, sem_i).wait()
        scs = [pltpu.async_copy(ones, acc_n.at[dbuf.at[j]], sem_s, add=True)
               for j in range(2)]
        for d in scs:
            d.wait()
        fire_d(lax.rem(b + 2, TBLKD), dbuf, sem_i)

    def outer_d(i2, carry):
        proc_d(2 * i2, di0, sem_i0)
        proc_d(2 * i2 + 1, di1, sem_i1)
        return carry

    lax.fori_loop(0, TBLKD // 2, outer_d, 0)
    for dbuf, sem in ((di0, sem_i0), (di1, sem_i1)):
        pltpu.make_async_copy(dst_hbm.at[pl.ds(0, 2)], dbuf, sem).wait()
    plsc.subcore_barrier()

    pltpu.sync_copy(acc_n.at[pl.ds(s * 3200, 3200)], db)
    pltpu.sync_copy(batch_hbm.at[pl.ds(s * 3200, 3200)], pb)

    def pack_step(i, carry):
        o = i * 16
        x = (db[pl.ds(o, 16)] + 1).astype(jnp.float32)
        ii = lax.bitcast_convert_type(x, jnp.int32)
        y = lax.bitcast_convert_type(
            jnp.int32(0x5F3759DF) - lax.shift_right_arithmetic(ii, 1),
            jnp.float32)
        for _ in range(3):
            y = y * (1.5 - 0.5 * x * y * y)
        db[pl.ds(o, 16)] = (
            (lax.bitcast_convert_type(y, jnp.int32) & jnp.int32(-8))
            | pb[pl.ds(o, 16)])
        return carry

    lax.fori_loop(0, 200, pack_step, 0)
    pltpu.sync_copy(db, acc_n.at[pl.ds(s * 3200, 3200)])

    @pl.when(s // 8 == c)
    def _():
        pltpu.sync_copy(db, pk_out.at[pl.ds(s * 3200, 3200)])
    plsc.subcore_barrier()

    basec = c * HALF + s * (2 * TBLK)

    def fire_c(b, sbuf, dbuf, sem):
        r = basec + b * 2
        pltpu.async_copy(src_hbm.at[pl.ds(r, 2)], sbuf, sem)
        pltpu.async_copy(dst_hbm.at[pl.ds(r, 2)], dbuf, sem)

    fire_c(0, si0, di0, sem_i0)
    fire_c(1, si1, di1, sem_i1)

    def proc_c(b, sbuf, dbuf, pv, wv, fx, sem_i):
        pltpu.make_async_copy(src_hbm.at[pl.ds(0, 2)], sbuf, sem_i).wait()
        pltpu.make_async_copy(dst_hbm.at[pl.ds(0, 2)], dbuf, sem_i).wait()
        gs = [pltpu.async_copy(acc_n.at[sbuf.at[j]], pv.at[j], sem_g)
              for j in range(2)]
        for d in gs:
            d.wait()
        for j in range(2):
            for k in range(8):
                p16 = pv[j, pl.ds(16 * k, 16)]
                d16 = dbuf[j, pl.ds(16 * k, 16)]
                fx[j, pl.ds(16 * k, 16)] = d16 * 8 + (p16 & jnp.int32(7))
                wv[j, pl.ds(16 * k, 16)] = lax.bitcast_convert_type(
                    p16 & jnp.int32(-8), jnp.float32)
        scs = [pltpu.async_copy(wv.at[j], acc_s.at[fx.at[j]],
                                sem_s, add=True)
               for j in range(2)]
        for d in scs:
            d.wait()
        fire_c(lax.rem(b + 2, TBLK), sbuf, dbuf, sem_i)

    def outer_c(i2, carry):
        proc_c(2 * i2, si0, di0, pv0, wv0, fx0, sem_i0)
        proc_c(2 * i2 + 1, si1, di1, pv1, wv1, fx1, sem_i1)
        return carry

    lax.fori_loop(0, TBLK // 2, outer_c, 0)
    for sbuf, dbuf, sem in ((si0, di0, sem_i0), (si1, di1, sem_i1)):
        pltpu.make_async_copy(src_hbm.at[pl.ds(0, 2)], sbuf, sem).wait()
        pltpu.make_async_copy(dst_hbm.at[pl.ds(0, 2)], dbuf, sem).wait()
    plsc.subcore_barrier()
    pltpu.sync_copy(acc_s.at[pl.ds(s * 25600, 25600)],
                    s_out.at[c, pl.ds(s * 25600, 25600)])


def _sc_prep(src2, dst2, batch_pad, zeros_ni, zeros_s):
    f = pl.kernel(
        _sc_prep_body,
        out_type=(jax.ShapeDtypeStruct((NC, NPAD * 8), jnp.float32),
                  jax.ShapeDtypeStruct((NPAD,), jnp.int32)),
        mesh=_mesh(),
        scratch_types=[
            pltpu.VMEM((2, 128), jnp.int32),
            pltpu.VMEM((2, 128), jnp.int32),
            pltpu.VMEM((2, 128), jnp.int32),
            pltpu.VMEM((2, 128), jnp.int32),
            pltpu.VMEM((128,), jnp.int32),
            pltpu.VMEM((3200,), jnp.int32),
            pltpu.VMEM((3200,), jnp.int32),
            pltpu.VMEM((2, 128), jnp.int32),
            pltpu.VMEM((2, 128), jnp.int32),
            pltpu.VMEM((2, 128), jnp.float32),
            pltpu.VMEM((2, 128), jnp.float32),
            pltpu.VMEM((2, 128), jnp.int32),
            pltpu.VMEM((2, 128), jnp.int32),
            pltpu.SemaphoreType.DMA,
            pltpu.SemaphoreType.DMA,
            pltpu.SemaphoreType.DMA,
            pltpu.SemaphoreType.DMA,
            pltpu.VMEM_SHARED((NPAD,), jnp.int32),
            pltpu.VMEM_SHARED((NPAD * 8,), jnp.float32),
        ])
    return f(src2, dst2, batch_pad, zeros_ni, zeros_s)


def _sc_conv2_body(src_hbm, dst_hbm, y_hbm, zeros_hbm, out_hbm,
                   si0, di0, si1, di1, si2, di2, si3, di3,
                   m0, m1, m2, m3,
                   sem_i0, sem_i1, sem_i2, sem_i3, sem_g, sem_s, acc):
    c = lax.axis_index("c")
    s = lax.axis_index("s")
    base = s * TROWS
    yc = y_hbm.at[c]
    pipes = ((si0, di0, m0, sem_i0), (si1, di1, m1, sem_i1),
             (si2, di2, m2, sem_i2), (si3, di3, m3, sem_i3))

    def fire_idx(b, sbuf, dbuf, sem):
        r = base + b
        pltpu.async_copy(src_hbm.at[r], sbuf, sem)
        pltpu.async_copy(dst_hbm.at[r], dbuf, sem)

    for k, (sbuf, dbuf, _, sem) in enumerate(pipes):
        fire_idx(k, sbuf, dbuf, sem)
    pltpu.sync_copy(zeros_hbm.at[pl.ds(s * 3200, 3200), :],
                    acc.at[pl.ds(s * 3200, 3200), :])
    plsc.subcore_barrier()

    def outer(i4, carry):
        for sbuf, dbuf, msg, sem in pipes:
            pltpu.make_async_copy(src_hbm.at[0], sbuf, sem).wait()
            pltpu.make_async_copy(dst_hbm.at[0], dbuf, sem).wait()
            pltpu.async_copy(yc.at[sbuf], msg, sem_g)
        scs = []
        for sbuf, dbuf, msg, sem in pipes:
            pltpu.make_async_copy(yc.at[sbuf], msg, sem_g).wait()
            scs.append(pltpu.async_copy(msg, acc.at[dbuf], sem_s,
                                        add=True))
        for k, (sbuf, dbuf, msg, sem) in enumerate(pipes):
            scs[k].wait()
            fire_idx(lax.rem(4 * i4 + k + 4, TROWS), sbuf, dbuf, sem)
        return carry

    lax.fori_loop(0, TROWS // 4, outer, 0)
    for sbuf, dbuf, _, sem in pipes:
        pltpu.make_async_copy(src_hbm.at[0], sbuf, sem).wait()
        pltpu.make_async_copy(dst_hbm.at[0], dbuf, sem).wait()
    plsc.subcore_barrier()
    pltpu.sync_copy(acc.at[pl.ds(s * 3200, 3200), :],
                    out_hbm.at[c, pl.ds(s * 3200, 3200), :])


def _sc_conv2(src2, dst2, y2, zeros_m):
    f = pl.kernel(
        _sc_conv2_body,
        out_type=jax.ShapeDtypeStruct((NC, NPAD, 32), jnp.float32),
        mesh=_mesh(),
        compiler_params=pltpu.CompilerParams(use_tc_tiling_on_sc=False),
        scratch_types=[
            pltpu.VMEM((128,), jnp.int32),
            pltpu.VMEM((128,), jnp.int32),
            pltpu.VMEM((128,), jnp.int32),
            pltpu.VMEM((128,), jnp.int32),
            pltpu.VMEM((128,), jnp.int32),
            pltpu.VMEM((128,), jnp.int32),
            pltpu.VMEM((128,), jnp.int32),
            pltpu.VMEM((128,), jnp.int32),
            pltpu.VMEM((128, 32), jnp.float32),
            pltpu.VMEM((128, 32), jnp.float32),
            pltpu.VMEM((128, 32), jnp.float32),
            pltpu.VMEM((128, 32), jnp.float32),
            pltpu.SemaphoreType.DMA,
            pltpu.SemaphoreType.DMA,
            pltpu.SemaphoreType.DMA,
            pltpu.SemaphoreType.DMA,
            pltpu.SemaphoreType.DMA,
            pltpu.SemaphoreType.DMA,
            pltpu.VMEM_SHARED((NPAD, 32), jnp.float32),
        ])
    return f(src2, dst2, y2, zeros_m)


def _tc_dense_body(S_ref, pk_ref, z_ref, Wz_ref, bz_ref,
                   W1_ref, b1_ref, W2_ref, b2_ref,
                   y2_ref, q_ref):
    bf = jnp.bfloat16
    f32 = jnp.float32
    zn = lax.dot_general(z_ref[...].astype(bf), Wz_ref[...].astype(bf),
                         (((1,), (1,)), ((), ())),
                         preferred_element_type=f32) + bz_ref[...]
    xw1d = lax.dot_general(zn.astype(bf), W1_ref[...].astype(bf),
                           (((1,), (1,)), ((), ())),
                           preferred_element_type=f32)
    S = S_ref[0] + S_ref[1]
    pk = pk_ref[0, 0, :]
    dv = lax.bitcast_convert_type(pk & jnp.int32(-8), f32)
    bt = pk & jnp.int32(7)
    oh = (bt[:, None] == lax.broadcasted_iota(jnp.int32, (RB, 5), 1)
          ).astype(jnp.float32)
    S5 = S[:, :5] + dv[:, None] * oh
    t = S5[:, 0:1] * xw1d[0:1, :]
    for b in range(1, 5):
        t = t + S5[:, b:b + 1] * xw1d[b:b + 1, :]
    out1 = dv[:, None] * t + b1_ref[...]
    h1 = jnp.maximum(out1, 0.0)
    xw2 = lax.dot_general(h1.astype(bf), W2_ref[...].astype(bf),
                          (((1,), (1,)), ((), ())),
                          preferred_element_type=f32)
    y = dv[:, None] * xw2
    q_ref[...] = dv[:, None] * y + b2_ref[...]
    y2_ref[0] = y[:, :32]
    y2_ref[1] = y[:, 32:]


def _tc_dense(S3, pk3, z, Wz, bz, W1, b1, W2, b2):
    return pl.pallas_call(
        _tc_dense_body,
        grid=(GD,),
        in_specs=[
            pl.BlockSpec((2, RB, 8), lambda i: (0, i, 0)),
            pl.BlockSpec((1, 1, RB), lambda i: (i, 0, 0)),
            pl.BlockSpec((NB, NL), lambda i: (0, 0)),
            pl.BlockSpec((NH, NL), lambda i: (0, 0)),
            pl.BlockSpec((NH,), lambda i: (0,)),
            pl.BlockSpec((NH, NH), lambda i: (0, 0)),
            pl.BlockSpec((NH,), lambda i: (0,)),
            pl.BlockSpec((NH, NH), lambda i: (0, 0)),
            pl.BlockSpec((NH,), lambda i: (0,)),
        ],
        out_specs=[
            pl.BlockSpec((2, RB, 32), lambda i: (0, i, 0)),
            pl.BlockSpec((RB, NH), lambda i: (i, 0)),
        ],
        out_shape=[
            jax.ShapeDtypeStruct((2, NPAD, 32), jnp.float32),
            jax.ShapeDtypeStruct((NPAD, NH), jnp.float32),
        ],
    )(S3, pk3, z, Wz, bz, W1, b1, W2, b2)


def _tc_final_body(seg_ref, q_ref, pk_ref, Wo_ref, bo_ref, o_ref):
    pk = pk_ref[0, 0, :]
    dv = lax.bitcast_convert_type(pk & jnp.int32(-8), jnp.float32)
    wo = Wo_ref[...]
    q = q_ref[...]
    h0 = jnp.maximum(dv[:, None] * seg_ref[0] + q[:, :32], 0.0)
    h1 = jnp.maximum(dv[:, None] * seg_ref[1] + q[:, 32:], 0.0)
    bf = jnp.bfloat16
    f32 = jnp.float32
    lin = (jnp.dot(h0.astype(bf), wo[0, :32].astype(bf),
                   preferred_element_type=f32)
           + jnp.dot(h1.astype(bf), wo[0, 32:].astype(bf),
                     preferred_element_type=f32) + bo_ref[0])
    o_ref[...] = jnp.tanh(lin).reshape(1, 1, RB)


def _tc_final(seg2, q, pk3, Wo, bo):
    return pl.pallas_call(
        _tc_final_body,
        grid=(GD,),
        in_specs=[
            pl.BlockSpec((2, RB, 32), lambda i: (0, i, 0)),
            pl.BlockSpec((RB, NH), lambda i: (i, 0)),
            pl.BlockSpec((1, 1, RB), lambda i: (i, 0, 0)),
            pl.BlockSpec((1, NH), lambda i: (0, 0)),
            pl.BlockSpec((1,), lambda i: (0,)),
        ],
        out_specs=pl.BlockSpec((1, 1, RB), lambda i: (i, 0, 0)),
        out_shape=jax.ShapeDtypeStruct((GD, 1, RB), jnp.float32),
    )(seg2, q, pk3, Wo, bo)


def kernel(z, edge_index, batch, Wz, bz, W1, b1, W2, b2, Wo, bo):
    npad_e = ROWS * 128 - NE
    pad = jnp.full((npad_e,), NT, jnp.int32)
    src2 = jnp.concatenate([edge_index[0], pad]).reshape(ROWS, 128)
    dst2 = jnp.concatenate([edge_index[1], pad]).reshape(ROWS, 128)
    zeros_ni = jnp.zeros((NPAD,), jnp.int32)
    zeros_s = jnp.zeros((NPAD * 8,), jnp.float32)
    zeros_m = jnp.zeros((NPAD, 32), jnp.float32)
    batch_pad = jnp.concatenate(
        [batch, jnp.zeros((NPAD - NT,), jnp.int32)])

    S2, packed = _sc_prep(src2, dst2, batch_pad, zeros_ni, zeros_s)
    S3 = S2.reshape(NC, NPAD, 8)
    pk3 = packed.reshape(GD, 1, RB)
    y2, q = _tc_dense(S3, pk3, z, Wz, bz, W1, b1, W2, b2)
    seg2 = _sc_conv2(src2, dst2, y2, zeros_m)
    spin3 = _tc_final(seg2, q, pk3, Wo, bo)
    return spin3.reshape(NPAD)[:NT].reshape(NB, NN)
